# Initial kernel scaffold; baseline (speedup 1.0000x reference)
import jax
import jax.numpy as jnp
from jax.experimental import pallas as pl


def kernel(x, edge_index, W1, b1, W2, b2):
    raise NotImplementedError("write your pallas kernel here")


# speedup vs baseline: 14.2493x; 14.2493x over previous
import functools

import jax
import jax.numpy as jnp
from jax import lax
from jax.experimental import pallas as pl
from jax.experimental.pallas import tpu as pltpu
from jax.experimental.pallas import tpu_sc as plsc

N = 10000
E = 320000
D = 128

NC = 2
NS = 16
NW = NC * NS
CHUNK = 128
CPW = -(-E // (NW * CHUNK))
E_PAD = NW * CPW * CHUNK
ACC_ROWS = 10240
STRIPE = ACC_ROWS // NS
ROW_BLK = 400
GRID = N // ROW_BLK

_mesh = plsc.VectorSubcoreMesh(
    core_axis_name="c", subcore_axis_name="s", num_cores=NC, num_subcores=NS
)


def _zero_stripe(zeros_hbm, buf, acc, s):
    pltpu.sync_copy(zeros_hbm, buf)
    for r in range(STRIPE // CHUNK):
        pltpu.sync_copy(buf, acc.at[pl.ds(s * STRIPE + r * CHUNK, CHUNK)])
    plsc.subcore_barrier()
    for r in range(STRIPE // CHUNK):
        pltpu.sync_copy(buf, acc.at[pl.ds(s * STRIPE + r * CHUNK, CHUNK)])
    plsc.subcore_barrier()


def _copy_out(acc, out_hbm, c, s):
    for r in range(STRIPE // CHUNK):
        off = s * STRIPE + r * CHUNK
        pltpu.sync_copy(acc.at[pl.ds(off, CHUNK)], out_hbm.at[c, pl.ds(off, CHUNK)])


@functools.partial(
    pl.kernel,
    out_type=jax.ShapeDtypeStruct((NC, ACC_ROWS, D), jnp.float32),
    mesh=_mesh,
    scratch_types=[
        pltpu.VMEM((CHUNK,), jnp.int32),
        pltpu.VMEM((CHUNK, D), jnp.float32),
        pltpu.VMEM((CHUNK, D), jnp.float32),
        pltpu.VMEM_SHARED((ACC_ROWS, D), jnp.float32),
    ],
)
def _deg_kernel(dst_hbm, ones_hbm, zeros_hbm, out_hbm, dst_c, ones_v, zbuf, acc):
    c = lax.axis_index("c")
    s = lax.axis_index("s")
    wid = s * NC + c
    pltpu.sync_copy(ones_hbm, ones_v)
    _zero_stripe(zeros_hbm, zbuf, acc, s)

    @pl.loop(0, CPW)
    def _(j):
        pltpu.sync_copy(dst_hbm.at[wid, j], dst_c)
        pltpu.sync_copy(ones_v, acc.at[dst_c], add=True)

    plsc.subcore_barrier()
    _copy_out(acc, out_hbm, c, s)


@functools.partial(
    pl.kernel,
    out_type=jax.ShapeDtypeStruct((NC, ACC_ROWS, D), jnp.float32),
    mesh=_mesh,
    scratch_types=[
        pltpu.VMEM((CHUNK,), jnp.int32),
        pltpu.VMEM((CHUNK,), jnp.int32),
        pltpu.VMEM((CHUNK, D), jnp.float32),
        pltpu.VMEM_SHARED((ACC_ROWS, D), jnp.float32),
        pltpu.SemaphoreType.DMA,
    ],
)
def _edge_kernel(y_hbm, src_hbm, dst_hbm, zeros_hbm, out_hbm,
                 src_c, dst_c, buf, acc, gsem):
    c = lax.axis_index("c")
    s = lax.axis_index("s")
    wid = s * NC + c
    _zero_stripe(zeros_hbm, buf, acc, s)

    @pl.loop(0, CPW)
    def _(j):
        pltpu.sync_copy(src_hbm.at[wid, j], src_c)
        pltpu.sync_copy(dst_hbm.at[wid, j], dst_c)
        pltpu.async_copy(y_hbm.at[src_c], buf, gsem).wait()
        pltpu.sync_copy(buf, acc.at[dst_c], add=True)

    plsc.subcore_barrier()
    _copy_out(acc, out_hbm, c, s)


def _dinv_from(degp):
    deg = degp[0, :, 0:1] + degp[1, :, 0:1] + 1.0
    return lax.rsqrt(deg)


def _tc1_body(degp_ref, x_ref, w_ref, y_ref):
    dinv = _dinv_from(degp_ref[...])
    xw = jnp.dot(x_ref[...], w_ref[...], preferred_element_type=jnp.float32)
    y_ref[...] = dinv * xw


def _tc2_body(degp_ref, zp_ref, y1_ref, b1_ref, w_ref, y2_ref):
    dinv = _dinv_from(degp_ref[...])
    h = dinv * (zp_ref[0] + zp_ref[1] + y1_ref[...]) + b1_ref[...]
    h = jnp.maximum(h, 0.0)
    y2_ref[...] = dinv * jnp.dot(h, w_ref[...], preferred_element_type=jnp.float32)


def _tc3_body(degp_ref, zp_ref, y2_ref, b2_ref, out_ref):
    dinv = _dinv_from(degp_ref[...])
    out_ref[...] = dinv * (zp_ref[0] + zp_ref[1] + y2_ref[...]) + b2_ref[...]


_pp_spec = pl.BlockSpec((NC, ROW_BLK, D), lambda i: (0, i, 0))
_row_spec = pl.BlockSpec((ROW_BLK, D), lambda i: (i, 0))
_w_spec = pl.BlockSpec((D, D), lambda i: (0, 0))
_b_spec = pl.BlockSpec((1, D), lambda i: (0, 0))
_row_out = jax.ShapeDtypeStruct((N, D), jnp.float32)


def _tc1(degp, x, w1):
    return pl.pallas_call(
        _tc1_body, grid=(GRID,),
        in_specs=[_pp_spec, _row_spec, _w_spec],
        out_specs=_row_spec, out_shape=_row_out,
    )(degp, x, w1)


def _tc2(degp, zp, y1, b1, w2):
    return pl.pallas_call(
        _tc2_body, grid=(GRID,),
        in_specs=[_pp_spec, _pp_spec, _row_spec, _b_spec, _w_spec],
        out_specs=_row_spec, out_shape=_row_out,
    )(degp, zp, y1, b1, w2)


def _tc3(degp, zp, y2, b2):
    return pl.pallas_call(
        _tc3_body, grid=(GRID,),
        in_specs=[_pp_spec, _pp_spec, _row_spec, _b_spec],
        out_specs=_row_spec, out_shape=_row_out,
    )(degp, zp, y2, b2)


def kernel(x, edge_index, W1, b1, W2, b2):
    ei = edge_index.astype(jnp.int32)
    pad = E_PAD - E
    pad_ids = jnp.arange(pad, dtype=jnp.int32)
    src3 = jnp.concatenate([ei[0], pad_ids % N]).reshape(NW, CPW, CHUNK)
    dst3 = jnp.concatenate([ei[1], N + pad_ids % (ACC_ROWS - N)]).reshape(
        NW, CPW, CHUNK
    )
    onesd = jnp.ones((CHUNK, D), jnp.float32)
    zerosd = jnp.zeros((CHUNK, D), jnp.float32)
    b1r = b1.reshape(1, D)
    b2r = b2.reshape(1, D)

    degp = _deg_kernel(dst3, onesd, zerosd)
    y1 = _tc1(degp, x, W1)
    z1 = _edge_kernel(y1, src3, dst3, zerosd)
    y2 = _tc2(degp, z1, y1, b1r, W2)
    z2 = _edge_kernel(y2, src3, dst3, zerosd)
    return _tc3(degp, z2, y2, b2r)


# speedup vs baseline: 22.7389x; 1.5958x over previous
import functools

import jax
import jax.numpy as jnp
from jax import lax
from jax.experimental import pallas as pl
from jax.experimental.pallas import tpu as pltpu
from jax.experimental.pallas import tpu_sc as plsc

N = 10000
E = 320000
D = 128

NC = 2
NS = 16
NW = NC * NS
CHUNK = 128
CPW = 80
E_PAD = NW * CPW * CHUNK
ACC_ROWS = 10240
STRIPE = ACC_ROWS // NS
ROW_BLK = 400
GRID = N // ROW_BLK

_mesh = plsc.VectorSubcoreMesh(
    core_axis_name="c", subcore_axis_name="s", num_cores=NC, num_subcores=NS
)


def _zero_stripe(zeros_hbm, buf, acc, s):
    pltpu.sync_copy(zeros_hbm, buf)
    for r in range(STRIPE // CHUNK):
        pltpu.sync_copy(buf, acc.at[pl.ds(s * STRIPE + r * CHUNK, CHUNK)])
    plsc.subcore_barrier()
    for r in range(STRIPE // CHUNK):
        pltpu.sync_copy(buf, acc.at[pl.ds(s * STRIPE + r * CHUNK, CHUNK)])


def _copy_out(acc, out_hbm, c, s):
    for r in range(STRIPE // CHUNK):
        off = s * STRIPE + r * CHUNK
        pltpu.sync_copy(acc.at[pl.ds(off, CHUNK)], out_hbm.at[c, pl.ds(off, CHUNK)])


@functools.partial(
    pl.kernel,
    out_type=jax.ShapeDtypeStruct((NC, ACC_ROWS, D), jnp.float32),
    mesh=_mesh,
    scratch_types=[
        pltpu.VMEM((CPW, CHUNK), jnp.int32),
        pltpu.VMEM((CHUNK, D), jnp.float32),
        pltpu.VMEM_SHARED((ACC_ROWS, D), jnp.float32),
        pltpu.SemaphoreType.DMA,
        pltpu.SemaphoreType.DMA,
    ],
)
def _deg_kernel(dst_hbm, ones_hbm, zeros_hbm, out_hbm,
                dst_v, ones_v, acc, s0, s1):
    c = lax.axis_index("c")
    s = lax.axis_index("s")
    wid = s * NC + c
    _zero_stripe(zeros_hbm, ones_v, acc, s)
    plsc.subcore_barrier()
    pltpu.sync_copy(dst_hbm.at[wid], dst_v)
    pltpu.sync_copy(ones_hbm, ones_v)
    plsc.subcore_barrier()

    @pl.loop(0, CPW, step=2)
    def _(j):
        @pl.when(j >= 2)
        def _():
            pltpu.make_async_copy(ones_v, acc.at[dst_v.at[j]], s0).wait()

        pltpu.async_copy(ones_v, acc.at[dst_v.at[j]], s0, add=True)

        @pl.when(j >= 2)
        def _():
            pltpu.make_async_copy(ones_v, acc.at[dst_v.at[j + 1]], s1).wait()

        pltpu.async_copy(ones_v, acc.at[dst_v.at[j + 1]], s1, add=True)

    pltpu.make_async_copy(ones_v, acc.at[dst_v.at[0]], s0).wait()
    pltpu.make_async_copy(ones_v, acc.at[dst_v.at[0]], s1).wait()
    plsc.subcore_barrier()
    _copy_out(acc, out_hbm, c, s)


@functools.partial(
    pl.kernel,
    out_type=jax.ShapeDtypeStruct((NC, ACC_ROWS, D), jnp.float32),
    mesh=_mesh,
    scratch_types=[
        pltpu.VMEM((CHUNK,), jnp.int32),
        pltpu.VMEM((CHUNK,), jnp.int32),
        pltpu.VMEM((CHUNK,), jnp.int32),
        pltpu.VMEM((CHUNK,), jnp.int32),
        pltpu.VMEM((CHUNK, D), jnp.float32),
        pltpu.VMEM((CHUNK, D), jnp.float32),
        pltpu.VMEM_SHARED((ACC_ROWS, D), jnp.float32),
        pltpu.SemaphoreType.DMA,
        pltpu.SemaphoreType.DMA,
        pltpu.SemaphoreType.DMA,
        pltpu.SemaphoreType.DMA,
        pltpu.SemaphoreType.DMA,
        pltpu.SemaphoreType.DMA,
        pltpu.SemaphoreType.DMA,
        pltpu.SemaphoreType.DMA,
    ],
)
def _edge_kernel(y_hbm, src_hbm, dst_hbm, zeros_hbm, out_hbm,
                 sib0, sib1, dib0, dib1, buf0, buf1, acc,
                 g0, g1, s0, s1, as0, as1, ad0, ad1):
    c = lax.axis_index("c")
    s = lax.axis_index("s")
    wid = s * NC + c
    _zero_stripe(zeros_hbm, buf0, acc, s)
    plsc.subcore_barrier()

    pltpu.async_copy(src_hbm.at[wid, 0], sib0, as0)
    pltpu.async_copy(src_hbm.at[wid, 1], sib1, as1)
    pltpu.async_copy(dst_hbm.at[wid, 0], dib0, ad0)
    pltpu.make_async_copy(src_hbm.at[wid, 0], sib0, as0).wait()
    pltpu.async_copy(y_hbm.at[sib0], buf0, g0)

    @pl.loop(0, CPW, step=2)
    def _(j):
        pltpu.make_async_copy(y_hbm.at[sib0], buf0, g0).wait()

        @pl.when(j + 2 < CPW)
        def _():
            pltpu.async_copy(src_hbm.at[wid, j + 2], sib0, as0)

        @pl.when(j >= 2)
        def _():
            pltpu.make_async_copy(buf1, acc.at[dib1], s1).wait()

        pltpu.async_copy(dst_hbm.at[wid, j + 1], dib1, ad1)
        pltpu.make_async_copy(src_hbm.at[wid, j + 1], sib1, as1).wait()
        pltpu.async_copy(y_hbm.at[sib1], buf1, g1)
        pltpu.make_async_copy(dst_hbm.at[wid, j], dib0, ad0).wait()
        pltpu.async_copy(buf0, acc.at[dib0], s0, add=True)
        pltpu.make_async_copy(y_hbm.at[sib1], buf1, g1).wait()

        @pl.when(j + 3 < CPW)
        def _():
            pltpu.async_copy(src_hbm.at[wid, j + 3], sib1, as1)

        pltpu.make_async_copy(buf0, acc.at[dib0], s0).wait()

        @pl.when(j + 2 < CPW)
        def _():
            pltpu.async_copy(dst_hbm.at[wid, j + 2], dib0, ad0)
            pltpu.make_async_copy(src_hbm.at[wid, j + 2], sib0, as0).wait()
            pltpu.async_copy(y_hbm.at[sib0], buf0, g0)

        pltpu.make_async_copy(dst_hbm.at[wid, j + 1], dib1, ad1).wait()
        pltpu.async_copy(buf1, acc.at[dib1], s1, add=True)

    pltpu.make_async_copy(buf1, acc.at[dib1], s1).wait()
    plsc.subcore_barrier()
    _copy_out(acc, out_hbm, c, s)


def _dinv_from(degp):
    deg = degp[0, :, 0:1] + degp[1, :, 0:1] + 1.0
    return lax.rsqrt(deg)


def _tc1_body(degp_ref, x_ref, w_ref, y_ref):
    dinv = _dinv_from(degp_ref[...])
    xw = jnp.dot(x_ref[...], w_ref[...], preferred_element_type=jnp.float32)
    y_ref[...] = dinv * xw


def _tc2_body(degp_ref, zp_ref, y1_ref, b1_ref, w_ref, y2_ref):
    dinv = _dinv_from(degp_ref[...])
    h = dinv * (zp_ref[0] + zp_ref[1] + y1_ref[...]) + b1_ref[...]
    h = jnp.maximum(h, 0.0)
    y2_ref[...] = dinv * jnp.dot(h, w_ref[...], preferred_element_type=jnp.float32)


def _tc3_body(degp_ref, zp_ref, y2_ref, b2_ref, out_ref):
    dinv = _dinv_from(degp_ref[...])
    out_ref[...] = dinv * (zp_ref[0] + zp_ref[1] + y2_ref[...]) + b2_ref[...]


_pp_spec = pl.BlockSpec((NC, ROW_BLK, D), lambda i: (0, i, 0))
_row_spec = pl.BlockSpec((ROW_BLK, D), lambda i: (i, 0))
_w_spec = pl.BlockSpec((D, D), lambda i: (0, 0))
_b_spec = pl.BlockSpec((1, D), lambda i: (0, 0))
_row_out = jax.ShapeDtypeStruct((N, D), jnp.float32)


def _tc1(degp, x, w1):
    return pl.pallas_call(
        _tc1_body, grid=(GRID,),
        in_specs=[_pp_spec, _row_spec, _w_spec],
        out_specs=_row_spec, out_shape=_row_out,
    )(degp, x, w1)


def _tc2(degp, zp, y1, b1, w2):
    return pl.pallas_call(
        _tc2_body, grid=(GRID,),
        in_specs=[_pp_spec, _pp_spec, _row_spec, _b_spec, _w_spec],
        out_specs=_row_spec, out_shape=_row_out,
    )(degp, zp, y1, b1, w2)


def _tc3(degp, zp, y2, b2):
    return pl.pallas_call(
        _tc3_body, grid=(GRID,),
        in_specs=[_pp_spec, _pp_spec, _row_spec, _b_spec],
        out_specs=_row_spec, out_shape=_row_out,
    )(degp, zp, y2, b2)


def kernel(x, edge_index, W1, b1, W2, b2):
    ei = edge_index.astype(jnp.int32)
    pad = E_PAD - E
    pad_ids = jnp.arange(pad, dtype=jnp.int32)
    src3 = jnp.concatenate([ei[0], pad_ids % N]).reshape(NW, CPW, CHUNK)
    dst3 = jnp.concatenate([ei[1], N + pad_ids % (ACC_ROWS - N)]).reshape(
        NW, CPW, CHUNK
    )
    onesd = jnp.ones((CHUNK, D), jnp.float32)
    zerosd = jnp.zeros((CHUNK, D), jnp.float32)
    b1r = b1.reshape(1, D)
    b2r = b2.reshape(1, D)

    degp = _deg_kernel(dst3, onesd, zerosd)
    y1 = _tc1(degp, x, W1)
    z1 = _edge_kernel(y1, src3, dst3, zerosd)
    y2 = _tc2(degp, z1, y1, b1r, W2)
    z2 = _edge_kernel(y2, src3, dst3, zerosd)
    return _tc3(degp, z2, y2, b2r)


# speedup vs baseline: 23.7767x; 1.0456x over previous
import functools

import jax
import jax.numpy as jnp
from jax import lax
from jax.experimental import pallas as pl
from jax.experimental.pallas import tpu as pltpu
from jax.experimental.pallas import tpu_sc as plsc

N = 10000
E = 320000
D = 128

NC = 2
NS = 16
NW = NC * NS
CHUNK = 128
CPW = 80
ECHUNK = 64
ECPW = 160
E_PAD = NW * CPW * CHUNK
ACC_ROWS = 10240
STRIPE = ACC_ROWS // NS
ROW_BLK = 400
GRID = N // ROW_BLK

_mesh = plsc.VectorSubcoreMesh(
    core_axis_name="c", subcore_axis_name="s", num_cores=NC, num_subcores=NS
)


def _zero_stripe(zeros_hbm, buf, acc, s):
    rows = buf.shape[0]
    pltpu.sync_copy(zeros_hbm, buf)
    for r in range(STRIPE // rows):
        pltpu.sync_copy(buf, acc.at[pl.ds(s * STRIPE + r * rows, rows)])
    plsc.subcore_barrier()
    for r in range(STRIPE // rows):
        pltpu.sync_copy(buf, acc.at[pl.ds(s * STRIPE + r * rows, rows)])


def _copy_out(acc, out_hbm, c, s):
    for r in range(STRIPE // CHUNK):
        off = s * STRIPE + r * CHUNK
        pltpu.sync_copy(acc.at[pl.ds(off, CHUNK)], out_hbm.at[c, pl.ds(off, CHUNK)])


@functools.partial(
    pl.kernel,
    out_type=jax.ShapeDtypeStruct((NC, ACC_ROWS, D), jnp.float32),
    mesh=_mesh,
    scratch_types=[
        pltpu.VMEM((CPW, CHUNK), jnp.int32),
        pltpu.VMEM((CHUNK, D), jnp.float32),
        pltpu.VMEM_SHARED((ACC_ROWS, D), jnp.float32),
        [pltpu.SemaphoreType.DMA for _ in range(4)],
    ],
)
def _deg_kernel(dst_hbm, ones_hbm, zeros_hbm, out_hbm,
                dst_v, ones_v, acc, sc):
    c = lax.axis_index("c")
    s = lax.axis_index("s")
    wid = s * NC + c
    _zero_stripe(zeros_hbm, ones_v, acc, s)
    plsc.subcore_barrier()
    pltpu.sync_copy(dst_hbm.at[wid], dst_v)
    pltpu.sync_copy(ones_hbm, ones_v)
    plsc.subcore_barrier()

    @pl.loop(0, CPW, step=4)
    def _(jbase):
        for b in range(4):
            j = jbase + b

            @pl.when(j >= 4)
            def _():
                pltpu.make_async_copy(ones_v, acc.at[dst_v.at[j]], sc[b]).wait()

            pltpu.async_copy(ones_v, acc.at[dst_v.at[j]], sc[b], add=True)

    for b in range(4):
        pltpu.make_async_copy(ones_v, acc.at[dst_v.at[b]], sc[b]).wait()
    plsc.subcore_barrier()
    _copy_out(acc, out_hbm, c, s)


@functools.partial(
    pl.kernel,
    out_type=jax.ShapeDtypeStruct((NC, ACC_ROWS, D), jnp.float32),
    mesh=_mesh,
    scratch_types=[
        [pltpu.VMEM((ECHUNK,), jnp.int32) for _ in range(4)],
        [pltpu.VMEM((ECHUNK,), jnp.int32) for _ in range(4)],
        [pltpu.VMEM((ECHUNK, D), jnp.float32) for _ in range(4)],
        pltpu.VMEM_SHARED((ACC_ROWS, D), jnp.float32),
        [pltpu.SemaphoreType.DMA for _ in range(4)],
        [pltpu.SemaphoreType.DMA for _ in range(4)],
        [pltpu.SemaphoreType.DMA for _ in range(4)],
        [pltpu.SemaphoreType.DMA for _ in range(4)],
    ],
)
def _edge_kernel(y_hbm, src_hbm, dst_hbm, zeros_hbm, out_hbm,
                 sidx, didx, h, acc, g, sc, asem, adsem):
    c = lax.axis_index("c")
    s = lax.axis_index("s")
    wid = s * NC + c
    _zero_stripe(zeros_hbm, h[0], acc, s)
    plsc.subcore_barrier()

    for r in range(4):
        pltpu.async_copy(src_hbm.at[wid, r], sidx[r], asem[r])
    for r in range(2):
        pltpu.async_copy(dst_hbm.at[wid, r], didx[r], adsem[r])
    for r in range(2):
        pltpu.make_async_copy(src_hbm.at[wid, r], sidx[r], asem[r]).wait()
        pltpu.async_copy(y_hbm.at[sidx[r]], h[r], g[r])

    @pl.loop(0, ECPW, step=4)
    def _(jbase):
        for b in range(4):
            j = jbase + b
            r = b
            q = (b + 2) % 4
            pltpu.make_async_copy(y_hbm.at[sidx[r]], h[r], g[r]).wait()

            @pl.when(j + 4 < ECPW)
            def _():
                pltpu.async_copy(src_hbm.at[wid, j + 4], sidx[r], asem[r])

            pltpu.make_async_copy(dst_hbm.at[wid, j], didx[r], adsem[r]).wait()
            pltpu.async_copy(h[r], acc.at[didx[r]], sc[r], add=True)

            @pl.when(j >= 2)
            def _():
                pltpu.make_async_copy(h[q], acc.at[didx[q]], sc[q]).wait()

            @pl.when(j + 2 < ECPW)
            def _():
                pltpu.async_copy(dst_hbm.at[wid, j + 2], didx[q], adsem[q])
                pltpu.make_async_copy(src_hbm.at[wid, j + 2], sidx[q], asem[q]).wait()
                pltpu.async_copy(y_hbm.at[sidx[q]], h[q], g[q])

    pltpu.make_async_copy(h[(ECPW - 2) % 4], acc.at[didx[(ECPW - 2) % 4]],
                          sc[(ECPW - 2) % 4]).wait()
    pltpu.make_async_copy(h[(ECPW - 1) % 4], acc.at[didx[(ECPW - 1) % 4]],
                          sc[(ECPW - 1) % 4]).wait()
    plsc.subcore_barrier()
    _copy_out(acc, out_hbm, c, s)


def _dinv_from(degp):
    deg = degp[0, :, 0:1] + degp[1, :, 0:1] + 1.0
    return lax.rsqrt(deg)


def _tc1_body(degp_ref, x_ref, w_ref, y_ref):
    dinv = _dinv_from(degp_ref[...])
    xw = jnp.dot(x_ref[...], w_ref[...], preferred_element_type=jnp.float32)
    y_ref[...] = dinv * xw


def _tc2_body(degp_ref, zp_ref, y1_ref, b1_ref, w_ref, y2_ref):
    dinv = _dinv_from(degp_ref[...])
    h = dinv * (zp_ref[0] + zp_ref[1] + y1_ref[...]) + b1_ref[...]
    h = jnp.maximum(h, 0.0)
    y2_ref[...] = dinv * jnp.dot(h, w_ref[...], preferred_element_type=jnp.float32)


def _tc3_body(degp_ref, zp_ref, y2_ref, b2_ref, out_ref):
    dinv = _dinv_from(degp_ref[...])
    out_ref[...] = dinv * (zp_ref[0] + zp_ref[1] + y2_ref[...]) + b2_ref[...]


_pp_spec = pl.BlockSpec((NC, ROW_BLK, D), lambda i: (0, i, 0))
_row_spec = pl.BlockSpec((ROW_BLK, D), lambda i: (i, 0))
_w_spec = pl.BlockSpec((D, D), lambda i: (0, 0))
_b_spec = pl.BlockSpec((1, D), lambda i: (0, 0))
_row_out = jax.ShapeDtypeStruct((N, D), jnp.float32)


def _tc1(degp, x, w1):
    return pl.pallas_call(
        _tc1_body, grid=(GRID,),
        in_specs=[_pp_spec, _row_spec, _w_spec],
        out_specs=_row_spec, out_shape=_row_out,
    )(degp, x, w1)


def _tc2(degp, zp, y1, b1, w2):
    return pl.pallas_call(
        _tc2_body, grid=(GRID,),
        in_specs=[_pp_spec, _pp_spec, _row_spec, _b_spec, _w_spec],
        out_specs=_row_spec, out_shape=_row_out,
    )(degp, zp, y1, b1, w2)


def _tc3(degp, zp, y2, b2):
    return pl.pallas_call(
        _tc3_body, grid=(GRID,),
        in_specs=[_pp_spec, _pp_spec, _row_spec, _b_spec],
        out_specs=_row_spec, out_shape=_row_out,
    )(degp, zp, y2, b2)


def kernel(x, edge_index, W1, b1, W2, b2):
    ei = edge_index.astype(jnp.int32)
    pad = E_PAD - E
    pad_ids = jnp.arange(pad, dtype=jnp.int32)
    src_flat = jnp.concatenate([ei[0], pad_ids % N])
    dst_flat = jnp.concatenate([ei[1], N + pad_ids % (ACC_ROWS - N)])
    src3 = src_flat.reshape(NW, CPW, CHUNK)
    dst3 = dst_flat.reshape(NW, CPW, CHUNK)
    src4 = src_flat.reshape(NW, ECPW, ECHUNK)
    dst4 = dst_flat.reshape(NW, ECPW, ECHUNK)
    onesd = jnp.ones((CHUNK, D), jnp.float32)
    zerosd = jnp.zeros((CHUNK, D), jnp.float32)
    zerose = jnp.zeros((ECHUNK, D), jnp.float32)
    b1r = b1.reshape(1, D)
    b2r = b2.reshape(1, D)

    degp = _deg_kernel(dst3, onesd, zerosd)
    y1 = _tc1(degp, x, W1)
    z1 = _edge_kernel(y1, src4, dst4, zerose)
    y2 = _tc2(degp, z1, y1, b1r, W2)
    z2 = _edge_kernel(y2, src4, dst4, zerose)
    return _tc3(degp, z2, y2, b2r)
